# baseline (speedup 1.0000x reference)
import functools

import jax
import jax.numpy as jnp
from jax import lax
from jax.experimental import pallas as pl
from jax.experimental.pallas import tpu as pltpu
from jax.experimental.pallas import tpu_sc as plsc


def _make_sc_copy(sl, dim, dtype):
    info = plsc.get_sparse_core_info()
    ns = info.num_subcores
    nw = info.num_cores * ns
    rows_per_w = sl // nw
    chunk = 64
    nchunks = rows_per_w // chunk
    mesh = plsc.VectorSubcoreMesh(core_axis_name="c", subcore_axis_name="s")

    @functools.partial(
        pl.kernel,
        mesh=mesh,
        out_type=jax.ShapeDtypeStruct((sl, dim), dtype),
        scratch_types=[
            pltpu.VMEM_SHARED((ns, chunk, dim), dtype),
            pltpu.VMEM_SHARED((ns, chunk, dim), dtype),
            pltpu.SemaphoreType.DMA,
            pltpu.SemaphoreType.DMA,
            pltpu.SemaphoreType.DMA,
            pltpu.SemaphoreType.DMA,
        ],
    )
    def sc_copy(table_hbm, out_hbm, shr0, shr1, isem0, isem1, osem0, osem1):
        sid = lax.axis_index("s")
        wid = sid * info.num_cores + lax.axis_index("c")
        base = wid * rows_per_w
        bufs = (shr0.at[sid], shr1.at[sid])
        isems = (isem0, isem1)
        osems = (osem0, osem1)

        def in_copy(i):
            b = i % 2
            return pltpu.make_async_copy(
                table_hbm.at[pl.ds(base + i * chunk, chunk)], bufs[b], isems[b]
            )

        def out_copy(i):
            b = i % 2
            return pltpu.make_async_copy(
                bufs[b], out_hbm.at[pl.ds(base + i * chunk, chunk)], osems[b]
            )

        in_copy(0).start()
        in_copy(1).start()
        for i in range(nchunks):
            in_copy(i).wait()
            out_copy(i).start()
            out_copy(i).wait()
            if i + 2 < nchunks:
                in_copy(i + 2).start()

    return sc_copy


def kernel(x, emb_weight):
    sl = x.shape[1]
    dim = emb_weight.shape[1]
    return _make_sc_copy(sl, dim, emb_weight.dtype)(emb_weight)
